# baseline (speedup 1.0000x reference)
import functools

import jax
import jax.numpy as jnp
from jax import lax
from jax.experimental import pallas as pl
from jax.experimental.pallas import tpu as pltpu
from jax.experimental.pallas import tpu_sc as plsc

_EPS = 1e-12
_MARGIN = 1.0
_L = 16
_CHUNK = 128


def _signed_unit(x):
    bits = lax.bitcast_convert_type(x, jnp.int32)
    one = jnp.int32(0x3F800000)
    sign_unit = lax.bitcast_convert_type(
        jnp.bitwise_or(jnp.bitwise_and(bits, jnp.int32(-0x80000000)), one),
        jnp.float32)
    return jnp.where(jnp.abs(x) >= _EPS, sign_unit, x * jnp.float32(1e12))


def _make_sc_kernel(nw, nch, d):
    mesh = plsc.VectorSubcoreMesh(core_axis_name="c", subcore_axis_name="s")
    info = plsc.get_sparse_core_info()
    nc = info.num_cores

    idx_t = pltpu.VMEM((nch, _CHUNK), jnp.int32)
    rows_t = pltpu.VMEM((nch, _CHUNK, d), jnp.float32)

    @functools.partial(
        pl.kernel,
        mesh=mesh,
        out_type=jax.ShapeDtypeStruct((nw, _L), jnp.float32),
        scratch_types=[idx_t] * 6 + [rows_t] * 6 + [
            pltpu.VMEM((_L,), jnp.float32),
            pltpu.SemaphoreType.DMA,
        ],
        compiler_params=pltpu.CompilerParams(use_tc_tiling_on_sc=False),
    )
    def sc_kernel(ph, pr, pt, nh, nr, nt, ent, rel, out,
                  phv, prv, ptv, nhv, nrv, ntv,
                  phr, prr, ptr, nhr, nrr, ntr,
                  accv, sem):
        wid = lax.axis_index("s") * nc + lax.axis_index("c")

        for src, dst in ((ph, phv), (pr, prv), (pt, ptv),
                         (nh, nhv), (nr, nrv), (nt, ntv)):
            pltpu.sync_copy(src.at[wid], dst)

        copies = []
        for j in range(nch):
            copies += [
                pltpu.async_copy(ent.at[phv.at[j]], phr.at[j], sem),
                pltpu.async_copy(rel.at[prv.at[j]], prr.at[j], sem),
                pltpu.async_copy(ent.at[ptv.at[j]], ptr.at[j], sem),
                pltpu.async_copy(ent.at[nhv.at[j]], nhr.at[j], sem),
                pltpu.async_copy(rel.at[nrv.at[j]], nrr.at[j], sem),
                pltpu.async_copy(ent.at[ntv.at[j]], ntr.at[j], sem),
            ]
        for cp in copies:
            cp.wait()

        def make_body(j):
            def body(i, acc):
                for c in range(0, d, _L):
                    sl = pl.ds(c, _L)
                    pos = jnp.abs(_signed_unit(phr[j, i, sl]) + prr[j, i, sl]
                                  - _signed_unit(ptr[j, i, sl]))
                    neg = jnp.abs(_signed_unit(nhr[j, i, sl]) + nrr[j, i, sl]
                                  - _signed_unit(ntr[j, i, sl]))
                    acc = acc + (pos - neg)
                return acc
            return body

        acc = jnp.zeros((_L,), jnp.float32)
        for j in range(nch):
            acc = lax.fori_loop(0, _CHUNK, make_body(j), acc)

        accv[...] = acc
        pltpu.sync_copy(accv, out.at[wid])

    return sc_kernel


def kernel(pos_exmpls, neg_exmpls, entity_emb, relation_emb):
    b, _ = pos_exmpls.shape
    _, d = entity_emb.shape
    info = plsc.get_sparse_core_info()
    nw = info.num_cores * info.num_subcores
    pb = b // nw
    nch = pb // _CHUNK

    def col(ex, c):
        return ex[:, c].reshape(nw, nch, _CHUNK).astype(jnp.int32)

    sc = _make_sc_kernel(nw, nch, d)
    partials = sc(col(pos_exmpls, 0), col(pos_exmpls, 1), col(pos_exmpls, 2),
                  col(neg_exmpls, 0), col(neg_exmpls, 1), col(neg_exmpls, 2),
                  entity_emb, relation_emb)
    return jnp.maximum(jnp.sum(partials) + jnp.float32(_MARGIN),
                       jnp.float32(0.0))
